# baseline (speedup 1.0000x reference)
import functools

import jax
import jax.numpy as jnp
from jax import lax
from jax.experimental import pallas as pl
from jax.experimental.pallas import tpu as pltpu
from jax.experimental.pallas import tpu_sc as plsc

VOCAB = 21
EMB = 128
BATCH = 16384
PEP_LEN = 21
MHC_LEN = 34
PEPTIDE_PAD = 3
TAB = 24

NC = 2
NS = 16
NW = NC * NS
WB = BATCH // NW


def _fill_chunk(c, cb, seq_len, pad, tab_v, idx_v, buf):
    rows = cb * seq_len

    @plsc.parallel_loop(0, rows // 16, unroll=2)
    def group(g):
        iv = idx_v[pl.ds(c * rows + g * 16, 16)]
        for p in range(16):
            base = iv[p] * 128
            r = g * 16 + p
            b = r // seq_len
            rr = b * pad + (r - b * seq_len)
            for k in range(8):
                buf[rr, pl.ds(16 * k, 16)] = tab_v[pl.ds(base + 16 * k, 16)]


def _sc_body(seq_len, cb, pad, idx_hbm, tab_hbm, out_hbm,
             tab_v, idx_v, buf0, buf1, sem0, sem1):
    wid = lax.axis_index("s") * NC + lax.axis_index("c")
    nchunks = WB // cb

    pltpu.sync_copy(tab_hbm, tab_v)
    pltpu.sync_copy(idx_hbm.at[pl.ds(wid * (WB * seq_len), WB * seq_len)],
                    idx_v)

    bufs = (buf0, buf1)
    sems = (sem0, sem1)

    def copies(c, slot):
        for b in range(cb):
            yield pltpu.make_async_copy(
                bufs[slot].at[pl.ds(b * pad, seq_len)],
                out_hbm.at[wid * WB + c * cb + b],
                sems[slot])

    def sup(c2, carry):
        a = 2 * c2
        b = a + 1

        @pl.when(c2 > 0)
        def _():
            for cp in copies(a, 0):
                cp.wait()
        _fill_chunk(a, cb, seq_len, pad, tab_v, idx_v, bufs[0])
        for cp in copies(a, 0):
            cp.start()

        @pl.when(c2 > 0)
        def _():
            for cp in copies(b, 1):
                cp.wait()
        _fill_chunk(b, cb, seq_len, pad, tab_v, idx_v, bufs[1])
        for cp in copies(b, 1):
            cp.start()
        return carry

    lax.fori_loop(0, nchunks // 2, sup, 0, unroll=False)
    for cp in copies(nchunks - 2, 0):
        cp.wait()
    for cp in copies(nchunks - 1, 1):
        cp.wait()


def _make_gather(seq_len, cb, pad):
    return pl.kernel(
        functools.partial(_sc_body, seq_len, cb, pad),
        out_type=jax.ShapeDtypeStruct((BATCH, seq_len, EMB), jnp.float32),
        mesh=plsc.VectorSubcoreMesh(core_axis_name="c", subcore_axis_name="s"),
        compiler_params=pltpu.CompilerParams(needs_layout_passes=False),
        scratch_types=[
            pltpu.VMEM((TAB * EMB,), jnp.float32),
            pltpu.VMEM((WB * seq_len,), jnp.int32),
            pltpu.VMEM((cb * pad, EMB), jnp.float32),
            pltpu.VMEM((cb * pad, EMB), jnp.float32),
            pltpu.SemaphoreType.DMA,
            pltpu.SemaphoreType.DMA,
        ],
    )


_pep_gather = _make_gather(PEP_LEN, 16, 24)
_mhc_gather = _make_gather(MHC_LEN, 8, 40)


def _mask_body(x_ref, o_ref):
    o_ref[...] = x_ref[...] != 0


_mask = pl.pallas_call(
    _mask_body,
    out_shape=jax.ShapeDtypeStruct((BATCH, PEP_LEN - 2 * PEPTIDE_PAD), jnp.bool_),
    grid=(8,),
    in_specs=[pl.BlockSpec((BATCH // 8, PEP_LEN - 2 * PEPTIDE_PAD),
                           lambda i: (i, 0))],
    out_specs=pl.BlockSpec((BATCH // 8, PEP_LEN - 2 * PEPTIDE_PAD),
                           lambda i: (i, 0)),
)


def kernel(peptide_x, mhc_x, peptide_emb_w, mhc_emb_w):
    pidx = peptide_x.astype(jnp.int32).reshape(BATCH * PEP_LEN)
    midx = mhc_x.astype(jnp.int32).reshape(BATCH * MHC_LEN)
    ptab = jnp.pad(peptide_emb_w, ((0, TAB - VOCAB), (0, 0))).reshape(TAB * EMB)
    mtab = jnp.pad(mhc_emb_w, ((0, TAB - VOCAB), (0, 0))).reshape(TAB * EMB)
    pep_emb = _pep_gather(pidx, ptab)
    mhc_emb = _mhc_gather(midx, mtab)
    masks = _mask(peptide_x[:, PEPTIDE_PAD:PEP_LEN - PEPTIDE_PAD].astype(jnp.int32))
    return pep_emb, mhc_emb, masks
